# baseline (speedup 1.0000x reference)
import jax
import jax.numpy as jnp
from jax.experimental import pallas as pl
from jax.experimental.pallas import tpu as pltpu

_OUT_D = 8192
_IN_D = 2048
_B = 128
_K = 164
_BOOST = 100.0
_JBLK = 1024
_NJ = _OUT_D // _JBLK


def _count_ge(u, thr):
    return jnp.sum((u >= thr).astype(jnp.int32), axis=1, keepdims=True)


def _hist16(blk, base):
    cnts = [_sum16((blk >= (base + b).astype(jnp.int16)).astype(jnp.int16))
            for b in range(1, 17)]
    return jnp.concatenate(cnts, axis=1)


def _sum16(x):
    w = x.shape[1]
    while w > 128:
        x = x[:, : w // 2] + x[:, w // 2 :]
        w //= 2
    return jnp.sum(x.astype(jnp.int32), axis=1, keepdims=True)


def _pooler_kernel(x_ref, conn_ref, avg_ref, out_ref, lo_ref, max_ref, h16_ref, l15_ref, s_ref):
    j = pl.program_id(0)
    avg = avg_ref[...]
    s = jnp.sum(avg)
    avg_blk = avg_ref[:, pl.ds(j * _JBLK, _JBLK)]
    neigh = (s - avg_blk) / (_OUT_D - 1)
    boost = jnp.exp(-_BOOST * (avg_blk - neigh))
    ov = jnp.dot(x_ref[...], conn_ref[...], preferred_element_type=jnp.float32)
    ovb = ov * boost
    out_ref[:, pl.ds(j * _JBLK, _JBLK)] = ovb

    ub = jax.lax.bitcast_convert_type(ovb, jnp.int32)
    bmax = jnp.max(ub, axis=1, keepdims=True)
    h16_ref[:, pl.ds(j * _JBLK, _JBLK)] = (ub >> 16).astype(jnp.int16)
    l15_ref[:, pl.ds(j * _JBLK, _JBLK)] = ((ub >> 1) & 0x7FFF).astype(jnp.int16)

    @pl.when(j == 0)
    def _seed():
        max_ref[...] = bmax

        ubh = (ub >> 16).astype(jnp.int16)

        def vbody(_, carry):
            lo, hi = carry
            mid = lo + jax.lax.div(hi - lo, 2)
            cnt = _sum16((ubh >= mid.astype(jnp.int16)).astype(jnp.int16))
            ge = cnt >= _K
            return jnp.where(ge, mid, lo), jnp.where(ge, hi, mid)

        lo0 = jnp.zeros((_B, 1), jnp.int32)
        hi0 = (bmax >> 16) + 1
        t0, _ = jax.lax.fori_loop(0, 15, vbody, (lo0, hi0))
        lo_ref[...] = t0 << 16
        s_ref[...] = jnp.zeros((_B, 16), jnp.int32)

    @pl.when(j > 0)
    def _accum_max():
        max_ref[...] = jnp.maximum(max_ref[...], bmax)
        prev = h16_ref[:, pl.ds((j - 1) * _JBLK, _JBLK)]
        base = lo_ref[...] >> 16
        s_ref[...] += _hist16(prev, base)

    @pl.when(j == _NJ - 1)
    def _select():
        u = jax.lax.bitcast_convert_type(out_ref[...], jnp.int32)

        h16 = h16_ref[...]
        base = lo_ref[...] >> 16
        s_tab = s_ref[...] + _hist16(
            h16_ref[:, pl.ds((_NJ - 1) * _JBLK, _JBLK)], base)
        noff = jnp.sum((s_tab >= _K).astype(jnp.int32), axis=1, keepdims=True)
        cols = jax.lax.broadcasted_iota(jnp.int32, (_B, 16), 1)
        s_next = jnp.sum(jnp.where(cols == noff, s_tab, 0), axis=1,
                         keepdims=True)
        in_win = noff < 16
        lo_a = base + noff
        hi_a = jnp.where(in_win, lo_a + 1, (max_ref[...] >> 16) + 1)
        ch_a = jnp.where(in_win, s_next, 0)

        def acond(carry):
            lo, hi, _ = carry
            return jnp.any(hi - lo > 1)

        def abody(carry):
            lo, hi, ch = carry
            mid = lo + jax.lax.div(hi - lo, 2)
            cnt = _sum16((h16 >= mid.astype(jnp.int16)).astype(jnp.int16))
            ge = cnt >= _K
            return (jnp.where(ge, mid, lo), jnp.where(ge, hi, mid),
                    jnp.where(ge, ch, cnt))

        t16, _, cnt_a = jax.lax.while_loop(
            acond, abody, (lo_a, hi_a, ch_a))
        k2 = _K - cnt_a

        eqv = jnp.where(h16 == t16.astype(jnp.int16), l15_ref[...],
                        jnp.int16(-1))

        def bcond(carry):
            lo, hi, _ = carry
            return jnp.any(hi - lo > 1)

        def bbody(carry):
            lo, hi, ch = carry
            mid = lo + jax.lax.div(hi - lo, 2)
            cnt = _sum16((eqv >= mid.astype(jnp.int16)).astype(jnp.int16))
            ge = cnt >= k2
            return (jnp.where(ge, mid, lo), jnp.where(ge, hi, mid),
                    jnp.where(ge, ch, cnt))

        t15, _, cnt_b = jax.lax.while_loop(
            bcond, bbody,
            (jnp.zeros((_B, 1), jnp.int32),
             jnp.full((_B, 1), 1 << 15, jnp.int32),
             jnp.zeros((_B, 1), jnp.int32)))

        base = (t16 << 16) | (t15 << 1)
        c1 = _count_ge(u, base + 1)
        up = c1 >= _K
        t = jnp.where(up, base + 1, base)
        c = jnp.where(up, cnt_a + cnt_b, c1)
        m = _K - c

        gt = u > t
        eq = u == t
        idx = jax.lax.broadcasted_iota(jnp.int32, (_B, _OUT_D), 1)

        def tcond(carry):
            need, _ = carry
            return jnp.max(need) > 0

        def tbody(carry):
            need, last = carry
            avail = eq & (idx > last)
            fi = jnp.min(jnp.where(avail, idx, _OUT_D), axis=1, keepdims=True)
            act = need > 0
            return need - act.astype(jnp.int32), jnp.where(act, fi, last)

        _, last = jax.lax.while_loop(
            tcond, tbody, (m, jnp.full((_B, 1), -1, jnp.int32)))

        out_ref[...] = (gt | (eq & (idx <= last))).astype(jnp.float32)


def kernel(x, connection, avg_activation):
    return pl.pallas_call(
        _pooler_kernel,
        grid=(_NJ,),
        in_specs=[
            pl.BlockSpec((_B, _IN_D), lambda j: (0, 0)),
            pl.BlockSpec((_IN_D, _JBLK), lambda j: (0, j)),
            pl.BlockSpec((1, _OUT_D), lambda j: (0, 0)),
        ],
        out_specs=pl.BlockSpec((_B, _OUT_D), lambda j: (0, 0)),
        out_shape=jax.ShapeDtypeStruct((_B, _OUT_D), jnp.float32),
        scratch_shapes=[
            pltpu.VMEM((_B, 1), jnp.int32),
            pltpu.VMEM((_B, 1), jnp.int32),
            pltpu.VMEM((_B, _OUT_D), jnp.int16),
            pltpu.VMEM((_B, _OUT_D), jnp.int16),
            pltpu.VMEM((_B, 16), jnp.int32),
        ],
    )(x, connection, avg_activation)


# speedup vs baseline: 1.1348x; 1.1348x over previous
import jax
import jax.numpy as jnp
from jax.experimental import pallas as pl
from jax.experimental.pallas import tpu as pltpu

_OUT_D = 8192
_IN_D = 2048
_B = 128
_K = 164
_BOOST = 100.0
_JBLK = 1024
_NJ = _OUT_D // _JBLK


def _count_ge(u, thr):
    return jnp.sum((u >= thr).astype(jnp.int32), axis=1, keepdims=True)


def _sum16(x):
    w = x.shape[1]
    while w > 128:
        x = x[:, : w // 2] + x[:, w // 2 :]
        w //= 2
    return jnp.sum(x.astype(jnp.int32), axis=1, keepdims=True)


def _pooler_kernel(x_ref, conn_ref, avg_ref, out_ref, lo_ref, max_ref, h16_ref, l16_ref):
    j = pl.program_id(0)
    avg = avg_ref[...]
    s = jnp.sum(avg)
    avg_blk = avg_ref[:, pl.ds(j * _JBLK, _JBLK)]
    neigh = (s - avg_blk) / (_OUT_D - 1)
    boost = jnp.exp(-_BOOST * (avg_blk - neigh))
    ov = jnp.dot(x_ref[...], conn_ref[...], preferred_element_type=jnp.float32)
    ovb = ov * boost
    out_ref[:, pl.ds(j * _JBLK, _JBLK)] = ovb

    ub = jax.lax.bitcast_convert_type(ovb, jnp.int32)
    bmax = jnp.max(ub, axis=1, keepdims=True)
    h16_ref[:, pl.ds(j * _JBLK, _JBLK)] = (ub >> 16).astype(jnp.int16)
    l16_ref[:, pl.ds(j * _JBLK, _JBLK)] = ((ub & 0xFFFF) ^ 0x8000).astype(
        jnp.int16)

    @pl.when(j == 0)
    def _seed():
        max_ref[...] = bmax

        ubh = (ub >> 16).astype(jnp.int16)

        def vbody(_, carry):
            lo, hi = carry
            mid = lo + jax.lax.div(hi - lo, 2)
            cnt = _sum16((ubh >= mid.astype(jnp.int16)).astype(jnp.int16))
            ge = cnt >= _K
            return jnp.where(ge, mid, lo), jnp.where(ge, hi, mid)

        lo0 = jnp.zeros((_B, 1), jnp.int32)
        hi0 = (bmax >> 16) + 1
        t0, _ = jax.lax.fori_loop(0, 15, vbody, (lo0, hi0))
        lo_ref[...] = t0 << 16

    @pl.when(j > 0)
    def _accum_max():
        max_ref[...] = jnp.maximum(max_ref[...], bmax)

    @pl.when(j == _NJ - 1)
    def _select():
        u = jax.lax.bitcast_convert_type(out_ref[...], jnp.int32)

        h16 = h16_ref[...]

        def acond(carry):
            lo, hi, _ = carry
            return jnp.any(hi - lo > 1)

        def abody(carry):
            lo, hi, ch = carry
            mid = lo + jax.lax.div(hi - lo, 2)
            cnt = _sum16((h16 >= mid.astype(jnp.int16)).astype(jnp.int16))
            ge = cnt >= _K
            return (jnp.where(ge, mid, lo), jnp.where(ge, hi, mid),
                    jnp.where(ge, ch, cnt))

        t16, _, cnt_a = jax.lax.while_loop(
            acond, abody,
            (lo_ref[...] >> 16, (max_ref[...] >> 16) + 1,
             jnp.zeros((_B, 1), jnp.int32)))
        k2 = _K - cnt_a

        eqv = jnp.where(h16 == t16.astype(jnp.int16), l16_ref[...],
                        jnp.int16(-(1 << 15)))

        def bcond(carry):
            lo, hi, _ = carry
            return jnp.any(hi - lo > 1)

        def bbody(carry):
            lo, hi, ch = carry
            mid = lo + jax.lax.div(hi - lo, 2)
            cnt = _sum16((eqv >= mid.astype(jnp.int16)).astype(jnp.int16))
            ge = cnt >= k2
            return (jnp.where(ge, mid, lo), jnp.where(ge, hi, mid),
                    jnp.where(ge, ch, cnt))

        tl, _, cnt_b = jax.lax.while_loop(
            bcond, bbody,
            (jnp.full((_B, 1), -(1 << 15), jnp.int32),
             jnp.full((_B, 1), 1 << 15, jnp.int32),
             jnp.zeros((_B, 1), jnp.int32)))

        t = (t16 << 16) | (tl + (1 << 15))
        m = _K - (cnt_a + cnt_b)

        gt = u > t
        eq = u == t
        idx = jax.lax.broadcasted_iota(jnp.int32, (_B, _OUT_D), 1)

        def tcond(carry):
            need, _ = carry
            return jnp.max(need) > 0

        def tbody(carry):
            need, last = carry
            avail = eq & (idx > last)
            fi = jnp.min(jnp.where(avail, idx, _OUT_D), axis=1, keepdims=True)
            act = need > 0
            return need - act.astype(jnp.int32), jnp.where(act, fi, last)

        _, last = jax.lax.while_loop(
            tcond, tbody, (m, jnp.full((_B, 1), -1, jnp.int32)))

        out_ref[...] = (gt | (eq & (idx <= last))).astype(jnp.float32)


def kernel(x, connection, avg_activation):
    return pl.pallas_call(
        _pooler_kernel,
        grid=(_NJ,),
        in_specs=[
            pl.BlockSpec((_B, _IN_D), lambda j: (0, 0)),
            pl.BlockSpec((_IN_D, _JBLK), lambda j: (0, j)),
            pl.BlockSpec((1, _OUT_D), lambda j: (0, 0)),
        ],
        out_specs=pl.BlockSpec((_B, _OUT_D), lambda j: (0, 0)),
        out_shape=jax.ShapeDtypeStruct((_B, _OUT_D), jnp.float32),
        scratch_shapes=[
            pltpu.VMEM((_B, 1), jnp.int32),
            pltpu.VMEM((_B, 1), jnp.int32),
            pltpu.VMEM((_B, _OUT_D), jnp.int16),
            pltpu.VMEM((_B, _OUT_D), jnp.int16),
        ],
    )(x, connection, avg_activation)
